# Initial kernel scaffold; baseline (speedup 1.0000x reference)
import jax
import jax.numpy as jnp
from jax.experimental import pallas as pl


def kernel(img, W1, gamma, beta, W2, b2):
    raise NotImplementedError("write your pallas kernel here")


# speedup vs baseline: 2.2887x; 2.2887x over previous
import functools

import numpy as np
import jax
import jax.numpy as jnp
from jax import lax
from jax.experimental import pallas as pl
from jax.experimental.pallas import tpu as pltpu
from jax.experimental.pallas import tpu_sc as plsc

_N, _C, _H, _W = 32, 3, 512, 512
_P = 16
_HP = _H // _P
_WP = _W // _P
_L = _HP * _WP
_MASK_RATIO = 0.75
_REMOVE = int(_L * _MASK_RATIO)
_KEEP = int(_L * (1 - _MASK_RATIO))
_NMOM = 9

_BF = jnp.bfloat16
_DOT = functools.partial(jnp.dot, preferred_element_type=jnp.float32)


def _split3(v):
    b1 = v.astype(_BF)
    r = v - b1.astype(jnp.float32)
    b2 = r.astype(_BF)
    r = r - b2.astype(jnp.float32)
    b3 = r.astype(_BF)
    return b1, b2, b3


def _stack_l(b1, b2, b3):
    return jnp.concatenate([b1, b1, b2, b1, b2, b3], axis=1)


def _stack_m(b1, b2, b3):
    return jnp.concatenate([b1, b2, b1, b3, b2, b1], axis=0)


def _cat0(x):
    return jnp.concatenate(_split3(x), axis=0)


def _cat1(x):
    return jnp.concatenate(_split3(x), axis=1)


def _dft_mats():
    idx = np.arange(_H, dtype=np.int64)
    jk = np.outer(idx, idx) % _H
    ang = 2.0 * np.pi * jk.astype(np.float64) / _H
    return (np.cos(ang).astype(np.float32), np.sin(ang).astype(np.float32))


def _pool_mats():
    sp = np.zeros((_H, _HP), dtype=np.float32)
    sp[np.arange(_H), np.arange(_H) // _P] = 1.0
    return sp.T.copy(), sp


def _pool(x, ste_ref, spe_ref):
    xs = _cat0(x)
    p = _DOT(ste_ref[...], xs)
    ps = _cat1(p)
    return _DOT(ps, spe_ref[...])


def _p1(img_ref, cl_ref, sl_ref, cr_ref, sr_ref, ste_ref, spe_ref,
        ol_ref, g_ref, mag_ref, e_ref, mom_ref):
    CL = cl_ref[...]
    SL = sl_ref[...]
    CR = cr_ref[...]
    SR = sr_ref[...]
    mags = []
    qs = []
    for ch in range(_C):
        X = img_ref[0, ch]
        Xs = _stack_m(*_split3(X))
        A = _DOT(CL, Xs)
        B = _DOT(SL, Xs)
        As = _stack_l(*_split3(A))
        Bs = _stack_l(*_split3(B))
        Re = _DOT(As, CR) - _DOT(Bs, SR)
        Im = _DOT(As, SR) + _DOT(Bs, CR)
        mg = jnp.sqrt(Re * Re + Im * Im) * (1.0 / 512.0)
        mag_ref[0, ch] = mg
        mags.append(mg)
        qs.append(_pool(mg, ste_ref, spe_ref))
    e_ref[0] = (qs[0] + qs[1] + qs[2]) * (1.0 / (_P * _P * _C))
    for a in range(_C):
        for b in range(a, _C):
            qs.append(_pool(mags[a] * mags[b], ste_ref, spe_ref))
    qq = jnp.concatenate(qs, axis=1)
    cs = _DOT(ol_ref[...], _cat0(qq))
    tot = _DOT(_cat1(cs), g_ref[...])
    mom_ref[0] = tot[0:1, :]


def _p2(par_ref, mag_ref, e_ref, ste_ref, spe_ref, out_ref):
    m0 = mag_ref[0, 0]
    m1 = mag_ref[0, 1]
    m2 = mag_ref[0, 2]
    fw = jnp.full((_H, _W), par_ref[5, 0], jnp.float32)
    for o in range(16):
        xo = (m0 * par_ref[0, o] + m1 * par_ref[1, o] + m2 * par_ref[2, o]
              + par_ref[3, o])
        fw = fw + jnp.maximum(xo, 0.0) * par_ref[4, o]
    fwp = _pool(fw, ste_ref, spe_ref) * (1.0 / (_P * _P))
    sig = 1.0 / (1.0 + jnp.exp(-fwp))
    out_ref[0] = e_ref[0] * sig


def _p3(e_ref, m_ref):
    ev = e_ref[0]
    R = jnp.broadcast_to(ev, (_L, _L))
    ri = lax.broadcasted_iota(jnp.int32, (_L, _L), 0)
    ci = lax.broadcasted_iota(jnp.int32, (_L, _L), 1)
    ecol = jnp.sum(jnp.where(ri == ci, R, 0.0), axis=1,
                   keepdims=True)
    gt = (R > ecol).astype(jnp.float32)
    tie = ((R == ecol) & (ci < ri)).astype(jnp.float32)
    rank = jnp.sum(gt + tie, axis=1, keepdims=True)
    m_ref[0] = (rank < _REMOVE).astype(jnp.float32)


def _sc_gather(mask_flat, idx_flat):
    info = plsc.get_sparse_core_info()
    nw = info.num_cores * info.num_subcores
    total = _N * _L
    chunk = total // nw
    mesh = plsc.VectorSubcoreMesh(core_axis_name="c", subcore_axis_name="s")

    @functools.partial(
        pl.kernel, mesh=mesh,
        compiler_params=pltpu.CompilerParams(needs_layout_passes=False),
        out_type=jax.ShapeDtypeStruct((total,), jnp.float32),
        scratch_types=[
            pltpu.VMEM((total,), jnp.float32),
            pltpu.VMEM((chunk,), jnp.int32),
            pltpu.VMEM((chunk,), jnp.float32),
        ],
    )
    def k(mask_hbm, idx_hbm, out_hbm, tbl_v, idx_v, out_v):
        wid = lax.axis_index("s") * info.num_cores + lax.axis_index("c")
        base = wid * chunk
        pltpu.sync_copy(mask_hbm, tbl_v)
        pltpu.sync_copy(idx_hbm.at[pl.ds(base, chunk)], idx_v)
        for v in range(chunk // 16):
            iv = idx_v[pl.ds(v * 16, 16)]
            out_v[pl.ds(v * 16, 16)] = plsc.load_gather(tbl_v, [iv])
        pltpu.sync_copy(out_v, out_hbm.at[pl.ds(base, chunk)])

    return k(mask_flat, idx_flat)


def _energy_rows(img, W1, gamma, beta, W2, b2):
    Cn, Sn = _dft_mats()
    Stn, Spn = _pool_mats()
    c1, c2, c3 = _split3(jnp.asarray(Cn))
    s1, s2, s3 = _split3(jnp.asarray(Sn))
    CL = _stack_l(c1, c2, c3)
    SL = _stack_l(s1, s2, s3)
    CR = _stack_m(c1, c2, c3)
    SR = _stack_m(s1, s2, s3)
    St = jnp.asarray(Stn).astype(_BF)
    Sp = jnp.asarray(Spn).astype(_BF)
    StE = jnp.concatenate([St, St, St], axis=1)
    SpE = jnp.concatenate([Sp, Sp, Sp], axis=0)
    OL = jnp.ones((8, 3 * _HP), _BF)
    gn = np.zeros((3 * _NMOM * _WP, 16), np.float32)
    cols = np.arange(3 * _NMOM * _WP)
    gn[cols, (cols % (_NMOM * _WP)) // _WP] = 1.0
    G = jnp.asarray(gn).astype(_BF)

    rep = lambda n: (0, 0)
    mag, e_grid, mom = pl.pallas_call(
        _p1,
        grid=(_N,),
        in_specs=[
            pl.BlockSpec((1, _C, _H, _W), lambda n: (n, 0, 0, 0)),
            pl.BlockSpec((_H, 6 * _H), rep),
            pl.BlockSpec((_H, 6 * _H), rep),
            pl.BlockSpec((6 * _H, _H), rep),
            pl.BlockSpec((6 * _H, _H), rep),
            pl.BlockSpec((_HP, 3 * _H), rep),
            pl.BlockSpec((3 * _H, _WP), rep),
            pl.BlockSpec((8, 3 * _HP), rep),
            pl.BlockSpec((3 * _NMOM * _WP, 16), rep),
        ],
        out_specs=[
            pl.BlockSpec((1, _C, _H, _W), lambda n: (n, 0, 0, 0)),
            pl.BlockSpec((1, _HP, _WP), lambda n: (n, 0, 0)),
            pl.BlockSpec((1, 1, 16), lambda n: (n, 0, 0)),
        ],
        out_shape=[
            jax.ShapeDtypeStruct((_N, _C, _H, _W), jnp.float32),
            jax.ShapeDtypeStruct((_N, _HP, _WP), jnp.float32),
            jax.ShapeDtypeStruct((_N, 1, 16), jnp.float32),
        ],
    )(img, CL, SL, CR, SR, StE, SpE, OL, G)

    s = mom.reshape(_N, 16).sum(axis=0) / float(_N * _H * _W)
    m3 = s[:3]
    M = jnp.stack([
        jnp.stack([s[3], s[4], s[5]]),
        jnp.stack([s[4], s[6], s[7]]),
        jnp.stack([s[5], s[7], s[8]]),
    ])
    mean16 = W1 @ m3
    ex2 = jnp.einsum('oc,cd,od->o', W1, M, W1)
    var16 = ex2 - mean16 * mean16
    a16 = gamma / jnp.sqrt(var16 + 1e-5)
    W1f = W1 * a16[:, None]
    b1f = beta - mean16 * a16
    par = jnp.stack([W1f[:, 0], W1f[:, 1], W1f[:, 2], b1f, W2[0],
                     jnp.full((16,), b2[0], jnp.float32)])

    e_final = pl.pallas_call(
        _p2,
        grid=(_N,),
        in_specs=[
            pl.BlockSpec(memory_space=pltpu.SMEM),
            pl.BlockSpec((1, _C, _H, _W), lambda n: (n, 0, 0, 0)),
            pl.BlockSpec((1, _HP, _WP), lambda n: (n, 0, 0)),
            pl.BlockSpec((_HP, 3 * _H), rep),
            pl.BlockSpec((3 * _H, _WP), rep),
        ],
        out_specs=pl.BlockSpec((1, _HP, _WP), lambda n: (n, 0, 0)),
        out_shape=jax.ShapeDtypeStruct((_N, _HP, _WP), jnp.float32),
    )(par, mag, e_grid, StE, SpE)

    return e_final.reshape(_N, 1, _L)


def kernel(img, W1, gamma, beta, W2, b2):
    Cn, Sn = _dft_mats()
    Stn, Spn = _pool_mats()
    c1, c2, c3 = _split3(jnp.asarray(Cn))
    s1, s2, s3 = _split3(jnp.asarray(Sn))
    CL = _stack_l(c1, c2, c3)
    SL = _stack_l(s1, s2, s3)
    CR = _stack_m(c1, c2, c3)
    SR = _stack_m(s1, s2, s3)
    St = jnp.asarray(Stn).astype(_BF)
    Sp = jnp.asarray(Spn).astype(_BF)
    StE = jnp.concatenate([St, St, St], axis=1)
    SpE = jnp.concatenate([Sp, Sp, Sp], axis=0)
    OL = jnp.ones((8, 3 * _HP), _BF)
    gn = np.zeros((3 * _NMOM * _WP, 16), np.float32)
    cols = np.arange(3 * _NMOM * _WP)
    gn[cols, (cols % (_NMOM * _WP)) // _WP] = 1.0
    G = jnp.asarray(gn).astype(_BF)
    rep = lambda n: (0, 0)
    mag, e_grid, mom = pl.pallas_call(
        _p1,
        grid=(_N,),
        in_specs=[
            pl.BlockSpec((1, _C, _H, _W), lambda n: (n, 0, 0, 0)),
            pl.BlockSpec((_H, 6 * _H), rep),
            pl.BlockSpec((_H, 6 * _H), rep),
            pl.BlockSpec((6 * _H, _H), rep),
            pl.BlockSpec((6 * _H, _H), rep),
            pl.BlockSpec((_HP, 3 * _H), rep),
            pl.BlockSpec((3 * _H, _WP), rep),
            pl.BlockSpec((8, 3 * _HP), rep),
            pl.BlockSpec((3 * _NMOM * _WP, 16), rep),
        ],
        out_specs=[
            pl.BlockSpec((1, _C, _H, _W), lambda n: (n, 0, 0, 0)),
            pl.BlockSpec((1, _HP, _WP), lambda n: (n, 0, 0)),
            pl.BlockSpec((1, 1, 16), lambda n: (n, 0, 0)),
        ],
        out_shape=[
            jax.ShapeDtypeStruct((_N, _C, _H, _W), jnp.float32),
            jax.ShapeDtypeStruct((_N, _HP, _WP), jnp.float32),
            jax.ShapeDtypeStruct((_N, 1, 16), jnp.float32),
        ],
    )(img, CL, SL, CR, SR, StE, SpE, OL, G)
    patches = mag.reshape(_N, _C, _HP, _P, _WP, _P).transpose(
        0, 1, 2, 4, 3, 5).reshape(_N, _C, _L, _P, _P)
    energy = patches.mean(axis=(3, 4)).mean(axis=1)
    x = jnp.einsum('nchw,oc->nohw', mag, W1)
    mean = x.mean(axis=(0, 2, 3))
    var = x.var(axis=(0, 2, 3))
    x = (x - mean[None, :, None, None]) / jnp.sqrt(
        var[None, :, None, None] + 1e-5)
    x = x * gamma[None, :, None, None] + beta[None, :, None, None]
    x = jax.nn.relu(x)
    fw = jnp.einsum('nchw,oc->nohw', x, W2) + b2[None, :, None, None]
    fwp = fw.reshape(_N, 1, _HP, _P, _WP, _P).transpose(
        0, 1, 2, 4, 3, 5).reshape(_N, 1, _L, _P * _P)
    freq_weights = fwp.mean(axis=3).reshape(_N, _L)
    freq_weights = jax.nn.sigmoid(freq_weights)
    energy = energy * freq_weights
    e_rows = energy.reshape(_N, 1, _L)
    mask_pre = pl.pallas_call(
        _p3,
        grid=(_N,),
        in_specs=[pl.BlockSpec((1, 1, _L), lambda n: (n, 0, 0))],
        out_specs=pl.BlockSpec((1, _L, 1), lambda n: (n, 0, 0)),
        out_shape=jax.ShapeDtypeStruct((_N, _L, 1), jnp.float32),
    )(e_rows)
    noise = jax.random.uniform(jax.random.key(42), (_N, _L),
                               dtype=jnp.float32)
    ids_shuffle = jnp.argsort(noise, axis=1)
    ids_restore = jnp.argsort(ids_shuffle, axis=1)
    ids_keep = ids_shuffle[:, :_KEEP]
    gidx = (jnp.arange(_N, dtype=jnp.int32)[:, None] * _L
            + ids_restore.astype(jnp.int32)).reshape(_N * _L)
    out_flat = _sc_gather(mask_pre.reshape(_N * _L), gidx)
    mask = out_flat.reshape(_N, _L)
    return (mask, ids_restore, ids_keep)
